# baseline (speedup 1.0000x reference)
import functools
import math

import jax
import jax.numpy as jnp
from jax import lax
from jax.experimental import pallas as pl
from jax.experimental.pallas import tpu as pltpu
from jax.experimental.pallas import tpu_sc as plsc

D_MODEL = 64
VOCAB = 1000000
SCALE = math.sqrt(D_MODEL)

NC = 2
NS = 16
NW = NC * NS

B_TOTAL = 4096 * 200
PER_W = B_TOTAL // NW
CH = 128
NCH = PER_W // CH
LANES = 16


def _sc_embed(x2d, lut):
    mesh = plsc.VectorSubcoreMesh(core_axis_name="c", subcore_axis_name="s")

    @functools.partial(
        pl.kernel,
        mesh=mesh,
        out_type=jax.ShapeDtypeStruct((B_TOTAL, D_MODEL), jnp.float32),
        scratch_types=[
            pltpu.VMEM((NCH, CH), jnp.int32),
            pltpu.VMEM((CH, D_MODEL), jnp.float32),
            pltpu.SemaphoreType.DMA,
        ],
        compiler_params=pltpu.CompilerParams(use_tc_tiling_on_sc=False),
    )
    def k(idx_hbm, table_hbm, out_hbm, idx_v, rows_v, gsem):
        cid = lax.axis_index("c")
        sid = lax.axis_index("s")
        wid = cid * NS + sid
        ibase = wid * NCH
        obase = wid * PER_W

        pltpu.sync_copy(idx_hbm.at[pl.ds(ibase, NCH)], idx_v)

        def chunk(j, carry):
            pltpu.async_copy(table_hbm.at[idx_v.at[j]], rows_v, gsem).wait()

            def srow(i, c):
                for t in range(D_MODEL // LANES):
                    sl = pl.ds(t * LANES, LANES)
                    rows_v[i, sl] = rows_v[i, sl] * SCALE
                return c

            lax.fori_loop(0, CH, srow, 0, unroll=2)
            pltpu.sync_copy(rows_v, out_hbm.at[pl.ds(obase + j * CH, CH)])
            return carry

        lax.fori_loop(0, NCH, chunk, 0)

    return k(x2d, lut)


def kernel(x, lut):
    x2d = x.reshape(B_TOTAL // CH, CH).astype(jnp.int32)
    out = _sc_embed(x2d, lut)
    return out.reshape(x.shape[0], x.shape[1], D_MODEL)


# speedup vs baseline: 1.1564x; 1.1564x over previous
import functools
import math

import jax
import jax.numpy as jnp
from jax import lax
from jax.experimental import pallas as pl
from jax.experimental.pallas import tpu as pltpu
from jax.experimental.pallas import tpu_sc as plsc

D_MODEL = 64
VOCAB = 1000000
SCALE = math.sqrt(D_MODEL)

NC = 2
NS = 16
NW = NC * NS

B_TOTAL = 4096 * 200
PER_W = B_TOTAL // NW
CH = 128
NCH = PER_W // CH
LANES = 16


NBUF = 4


def _sc_embed(x2d, lut):
    mesh = plsc.VectorSubcoreMesh(core_axis_name="c", subcore_axis_name="s")

    @functools.partial(
        pl.kernel,
        mesh=mesh,
        out_type=jax.ShapeDtypeStruct((B_TOTAL, D_MODEL), jnp.float32),
        scratch_types=[
            pltpu.VMEM((NCH, CH), jnp.int32),
            *[pltpu.VMEM((CH, D_MODEL), jnp.float32) for _ in range(NBUF)],
            *[pltpu.SemaphoreType.DMA for _ in range(NBUF)],
        ],
        compiler_params=pltpu.CompilerParams(use_tc_tiling_on_sc=False),
    )
    def k(idx_hbm, table_hbm, out_hbm, idx_v, *bufs_sems):
        bufs = bufs_sems[:NBUF]
        sems = bufs_sems[NBUF:]
        cid = lax.axis_index("c")
        sid = lax.axis_index("s")
        wid = cid * NS + sid
        ibase = wid * NCH
        obase = wid * PER_W

        pltpu.sync_copy(idx_hbm.at[pl.ds(ibase, NCH)], idx_v)

        for b in range(NBUF):
            pltpu.async_copy(table_hbm.at[idx_v.at[b]], bufs[b], sems[b])

        def step(g, carry):
            j0 = g * NBUF
            for b in range(NBUF):
                j = j0 + b
                buf = bufs[b]
                pltpu.make_async_copy(table_hbm.at[idx_v.at[j]], buf, sems[b]).wait()

                def srow(i, c, buf=buf):
                    for t in range(D_MODEL // LANES):
                        sl = pl.ds(t * LANES, LANES)
                        buf[i, sl] = buf[i, sl] * SCALE
                    return c

                lax.fori_loop(0, CH, srow, 0, unroll=8)
                pltpu.sync_copy(buf, out_hbm.at[pl.ds(obase + j * CH, CH)])

                nxt = j + NBUF

                @pl.when(nxt < NCH)
                def _(b=b, nxt=nxt, buf=buf):
                    pltpu.async_copy(table_hbm.at[idx_v.at[nxt]], buf, sems[b])

            return carry

        lax.fori_loop(0, NCH // NBUF, step, 0)

    return k(x2d, lut)


def kernel(x, lut):
    x2d = x.reshape(B_TOTAL // CH, CH).astype(jnp.int32)
    out = _sc_embed(x2d, lut)
    return out.reshape(x.shape[0], x.shape[1], D_MODEL)
